# Initial kernel scaffold; baseline (speedup 1.0000x reference)
import jax
import jax.numpy as jnp
from jax.experimental import pallas as pl


def kernel(x, edge_index, W1l, b1l, W1r, W2l, b2l, W2r):
    raise NotImplementedError("write your pallas kernel here")


# speedup vs baseline: 4.1642x; 4.1642x over previous
import functools

import jax
import jax.numpy as jnp
from jax import lax
from jax.experimental import pallas as pl
from jax.experimental.pallas import tpu as pltpu
from jax.experimental.pallas import tpu_sc as plsc

N = 10000
E = 320000
D = 128

NC = 2
NS = 16
NHALF = 5120
NPADROWS = 128
NROWS = NHALF + NPADROWS
ZPS = NROWS // NS
WPS = NHALF // NS
BATCH = 128
NB = -(-E // (NS * BATCH))
EPS = NB * BATCH
CNT_W = 16
NPOOL = NC * NHALF


def _sc_scatter_body(first, src_hbm, dst_hbm, x_hbm, *refs):
    if first:
        (acc_out, cnt_out, dstl_out, src_v, dst_v, rows_v, ones_v, zc_v,
         acc_s, cnt_s) = refs
    else:
        acc_out, src_v, dst_v, rows_v, acc_s = refs
    cid = lax.axis_index("c")
    sid = lax.axis_index("s")

    @pl.loop(0, BATCH)
    def _(i):
        @pl.loop(0, D, step=16)
        def _(j):
            rows_v[i, pl.ds(j, 16)] = jnp.zeros((16,), jnp.float32)

    if first:
        @pl.loop(0, BATCH)
        def _(i):
            ones_v[i, pl.ds(0, CNT_W)] = jnp.ones((CNT_W,), jnp.float32)

        @pl.loop(0, ZPS)
        def _(i):
            zc_v[i, pl.ds(0, CNT_W)] = jnp.zeros((CNT_W,), jnp.float32)

    zbase = sid * ZPS
    for off, size in ((0, BATCH), (BATCH, BATCH), (2 * BATCH, ZPS - 2 * BATCH)):
        pltpu.sync_copy(rows_v.at[pl.ds(0, size), :],
                        acc_s.at[pl.ds(zbase + off, size), :])
    if first:
        pltpu.sync_copy(zc_v, cnt_s.at[pl.ds(zbase, ZPS), :])

    plsc.subcore_barrier()

    pltpu.sync_copy(src_hbm.at[sid], src_v)
    if first:
        pltpu.sync_copy(dst_hbm.at[sid], dst_v)
        trash = NHALF + sid

        @pl.loop(0, NB)
        def _(i):
            @pl.loop(0, BATCH, step=16)
            def _(k):
                d = dst_v[i, pl.ds(k, 16)] - cid * NHALF
                ok = (d >= 0) & (d < NHALF)
                dst_v[i, pl.ds(k, 16)] = jnp.where(ok, d, trash)

        pltpu.sync_copy(dst_v, dstl_out.at[cid, sid])
    else:
        pltpu.sync_copy(dst_hbm.at[cid, sid], dst_v)

    @pl.loop(0, NB)
    def _(j):
        pltpu.sync_copy(x_hbm.at[src_v.at[j]], rows_v)
        pltpu.sync_copy(rows_v, acc_s.at[dst_v.at[j]], add=True)
        if first:
            pltpu.sync_copy(ones_v, cnt_s.at[dst_v.at[j]], add=True)

    plsc.subcore_barrier()

    wbase = sid * WPS
    pltpu.sync_copy(acc_s.at[pl.ds(wbase, WPS), :],
                    acc_out.at[cid, pl.ds(wbase, WPS), :])
    if first:
        pltpu.sync_copy(cnt_s.at[pl.ds(wbase, WPS), :],
                        cnt_out.at[cid, pl.ds(wbase, WPS), :])


def _make_sc_scatter(first):
    mesh = plsc.VectorSubcoreMesh(core_axis_name="c", subcore_axis_name="s")
    out_type = [jax.ShapeDtypeStruct((NC, NHALF, D), jnp.float32)]
    scratch = [
        pltpu.VMEM((NB, BATCH), jnp.int32),
        pltpu.VMEM((NB, BATCH), jnp.int32),
        pltpu.VMEM((BATCH, D), jnp.float32),
    ]
    if first:
        out_type += [
            jax.ShapeDtypeStruct((NC, NHALF, CNT_W), jnp.float32),
            jax.ShapeDtypeStruct((NC, NS, NB, BATCH), jnp.int32),
        ]
        scratch += [
            pltpu.VMEM((BATCH, CNT_W), jnp.float32),
            pltpu.VMEM((ZPS, CNT_W), jnp.float32),
        ]
    scratch.append(pltpu.VMEM_SHARED((NROWS, D), jnp.float32))
    if first:
        scratch.append(pltpu.VMEM_SHARED((NROWS, CNT_W), jnp.float32))
    return pl.kernel(functools.partial(_sc_scatter_body, first),
                     out_type=out_type, mesh=mesh, scratch_types=scratch,
                     compiler_params=pltpu.CompilerParams(
                         use_tc_tiling_on_sc=False))


def _tc_body(p_ref, c_ref, x_ref, wl_ref, bl_ref, wr_ref, o_ref):
    cnt = c_ref[0, :, 0:1]
    mean = p_ref[0] / jnp.maximum(cnt, 1.0)
    dn = (((1,), (1,)), ((), ()))
    acc = lax.dot_general(mean, wl_ref[...], dn,
                          preferred_element_type=jnp.float32)
    acc += lax.dot_general(x_ref[...], wr_ref[...], dn,
                           preferred_element_type=jnp.float32)
    o_ref[...] = jnp.maximum(acc + bl_ref[...], 0.0)


def _tc_layer(p, c, x, Wl, bl, Wr):
    BLK = 1280
    nblk = NHALF // BLK
    return pl.pallas_call(
        _tc_body,
        grid=(NPOOL // BLK,),
        in_specs=[
            pl.BlockSpec((1, BLK, D), lambda i: (i // nblk, i % nblk, 0)),
            pl.BlockSpec((1, BLK, CNT_W), lambda i: (i // nblk, i % nblk, 0)),
            pl.BlockSpec((BLK, D), lambda i: (i, 0)),
            pl.BlockSpec((D, D), lambda i: (0, 0)),
            pl.BlockSpec((1, D), lambda i: (0, 0)),
            pl.BlockSpec((D, D), lambda i: (0, 0)),
        ],
        out_specs=pl.BlockSpec((BLK, D), lambda i: (i, 0)),
        out_shape=jax.ShapeDtypeStruct((NPOOL, D), jnp.float32),
    )(p, c, x, Wl, bl, Wr)


def kernel(x, edge_index, W1l, b1l, W1r, W2l, b2l, W2r):
    src = edge_index[0].astype(jnp.int32)
    dst = edge_index[1].astype(jnp.int32)
    pad = NS * EPS - E
    src_p = jnp.concatenate([src, jnp.zeros((pad,), jnp.int32)])
    dst_p = jnp.concatenate([dst, jnp.full((pad,), NPOOL, jnp.int32)])
    src_p = src_p.reshape(NS, NB, BATCH)
    dst_p = dst_p.reshape(NS, NB, BATCH)
    x_p = jnp.pad(x, ((0, NPOOL - N), (0, 0)))
    b1 = b1l.reshape(1, D)
    b2 = b2l.reshape(1, D)

    p1, cnt, dstl = _make_sc_scatter(True)(src_p, dst_p, x_p)
    h = _tc_layer(p1, cnt, x_p, W1l, b1, W1r)
    (p2,) = _make_sc_scatter(False)(src_p, dstl, h)
    out = _tc_layer(p2, cnt, h, W2l, b2, W2r)
    return out[:N]


# speedup vs baseline: 5.1844x; 1.2450x over previous
import functools

import jax
import jax.numpy as jnp
from jax import lax
from jax.experimental import pallas as pl
from jax.experimental.pallas import tpu as pltpu
from jax.experimental.pallas import tpu_sc as plsc

N = 10000
E = 320000
D = 128

NC = 2
NS = 16
NHALF = 5120
NPADROWS = 128
NROWS = NHALF + NPADROWS
ZPS = NROWS // NS
WPS = NHALF // NS
BATCH = 128
NB = -(-E // (NS * BATCH))
EPS = NB * BATCH
CNT_W = 16
NPOOL = NC * NHALF


def _sc_scatter_body(first, src_hbm, dst_hbm, x_hbm, *refs):
    if first:
        (acc_out, cnt_out, dstl_out, src_v, dst_v, rows_v, ones_v, zc_v,
         acc_s, cnt_s, gsem) = refs
    else:
        acc_out, src_v, dst_v, rows_v, acc_s, gsem = refs
    cid = lax.axis_index("c")
    sid = lax.axis_index("s")

    @pl.loop(0, BATCH)
    def _(i):
        @pl.loop(0, D, step=16)
        def _(j):
            rows_v[0, i, pl.ds(j, 16)] = jnp.zeros((16,), jnp.float32)

    if first:
        @pl.loop(0, BATCH)
        def _(i):
            ones_v[i, pl.ds(0, CNT_W)] = jnp.ones((CNT_W,), jnp.float32)

        @pl.loop(0, ZPS)
        def _(i):
            zc_v[i, pl.ds(0, CNT_W)] = jnp.zeros((CNT_W,), jnp.float32)

    zbase = sid * ZPS
    for off, size in ((0, BATCH), (BATCH, BATCH), (2 * BATCH, ZPS - 2 * BATCH)):
        pltpu.sync_copy(rows_v.at[0, pl.ds(0, size), :],
                        acc_s.at[pl.ds(zbase + off, size), :])
    if first:
        pltpu.sync_copy(zc_v, cnt_s.at[pl.ds(zbase, ZPS), :])

    plsc.subcore_barrier()

    pltpu.sync_copy(src_hbm.at[sid], src_v)
    if first:
        pltpu.sync_copy(dst_hbm.at[sid], dst_v)
        trash = NHALF + sid * 8 + lax.rem(lax.iota(jnp.int32, 16), 8)

        @pl.loop(0, NB)
        def _(i):
            @pl.loop(0, BATCH, step=16)
            def _(k):
                d = dst_v[i, pl.ds(k, 16)] - cid * NHALF
                ok = (d >= 0) & (d < NHALF)
                dst_v[i, pl.ds(k, 16)] = jnp.where(ok, d, trash)

        pltpu.sync_copy(dst_v, dstl_out.at[cid, sid])
    else:
        pltpu.sync_copy(dst_hbm.at[cid, sid], dst_v)

    pltpu.async_copy(x_hbm.at[src_v.at[0]], rows_v.at[0], gsem)

    @pl.loop(0, NB)
    def _(j):
        par = lax.rem(j, 2)
        pltpu.make_async_copy(x_hbm.at[src_v.at[j]], rows_v.at[par],
                              gsem).wait()

        @pl.when(j + 1 < NB)
        def _():
            pltpu.async_copy(x_hbm.at[src_v.at[j + 1]],
                             rows_v.at[1 - par], gsem)

        pltpu.sync_copy(rows_v.at[par], acc_s.at[dst_v.at[j]], add=True)
        if first:
            pltpu.sync_copy(ones_v, cnt_s.at[dst_v.at[j]], add=True)

    plsc.subcore_barrier()

    wbase = sid * WPS
    pltpu.sync_copy(acc_s.at[pl.ds(wbase, WPS), :],
                    acc_out.at[cid, pl.ds(wbase, WPS), :])
    if first:
        pltpu.sync_copy(cnt_s.at[pl.ds(wbase, WPS), :],
                        cnt_out.at[cid, pl.ds(wbase, WPS), :])


def _make_sc_scatter(first):
    mesh = plsc.VectorSubcoreMesh(core_axis_name="c", subcore_axis_name="s")
    out_type = [jax.ShapeDtypeStruct((NC, NHALF, D), jnp.float32)]
    scratch = [
        pltpu.VMEM((NB, BATCH), jnp.int32),
        pltpu.VMEM((NB, BATCH), jnp.int32),
        pltpu.VMEM((2, BATCH, D), jnp.float32),
    ]
    if first:
        out_type += [
            jax.ShapeDtypeStruct((NC, NHALF, CNT_W), jnp.float32),
            jax.ShapeDtypeStruct((NC, NS, NB, BATCH), jnp.int32),
        ]
        scratch += [
            pltpu.VMEM((BATCH, CNT_W), jnp.float32),
            pltpu.VMEM((ZPS, CNT_W), jnp.float32),
        ]
    scratch.append(pltpu.VMEM_SHARED((NROWS, D), jnp.float32))
    if first:
        scratch.append(pltpu.VMEM_SHARED((NROWS, CNT_W), jnp.float32))
    scratch.append(pltpu.SemaphoreType.DMA)
    return pl.kernel(functools.partial(_sc_scatter_body, first),
                     out_type=out_type, mesh=mesh, scratch_types=scratch,
                     compiler_params=pltpu.CompilerParams(
                         use_tc_tiling_on_sc=False))


def _tc_body(p_ref, c_ref, x_ref, wl_ref, bl_ref, wr_ref, o_ref):
    cnt = c_ref[0, :, 0:1]
    mean = p_ref[0] / jnp.maximum(cnt, 1.0)
    dn = (((1,), (1,)), ((), ()))
    acc = lax.dot_general(mean, wl_ref[...], dn,
                          preferred_element_type=jnp.float32)
    acc += lax.dot_general(x_ref[...], wr_ref[...], dn,
                           preferred_element_type=jnp.float32)
    o_ref[...] = jnp.maximum(acc + bl_ref[...], 0.0)


def _tc_layer(p, c, x, Wl, bl, Wr):
    BLK = 1280
    nblk = NHALF // BLK
    return pl.pallas_call(
        _tc_body,
        grid=(NPOOL // BLK,),
        in_specs=[
            pl.BlockSpec((1, BLK, D), lambda i: (i // nblk, i % nblk, 0)),
            pl.BlockSpec((1, BLK, CNT_W), lambda i: (i // nblk, i % nblk, 0)),
            pl.BlockSpec((BLK, D), lambda i: (i, 0)),
            pl.BlockSpec((D, D), lambda i: (0, 0)),
            pl.BlockSpec((1, D), lambda i: (0, 0)),
            pl.BlockSpec((D, D), lambda i: (0, 0)),
        ],
        out_specs=pl.BlockSpec((BLK, D), lambda i: (i, 0)),
        out_shape=jax.ShapeDtypeStruct((NPOOL, D), jnp.float32),
    )(p, c, x, Wl, bl, Wr)


def kernel(x, edge_index, W1l, b1l, W1r, W2l, b2l, W2r):
    src = edge_index[0].astype(jnp.int32)
    dst = edge_index[1].astype(jnp.int32)
    pad = NS * EPS - E
    src_p = jnp.concatenate([src, jnp.zeros((pad,), jnp.int32)])
    dst_p = jnp.concatenate([dst, jnp.full((pad,), NPOOL, jnp.int32)])
    src_p = src_p.reshape(NS, NB, BATCH)
    dst_p = dst_p.reshape(NS, NB, BATCH)
    x_p = jnp.pad(x, ((0, NPOOL - N), (0, 0)))
    b1 = b1l.reshape(1, D)
    b2 = b2l.reshape(1, D)

    p1, cnt, dstl = _make_sc_scatter(True)(src_p, dst_p, x_p)
    h = _tc_layer(p1, cnt, x_p, W1l, b1, W1r)
    (p2,) = _make_sc_scatter(False)(src_p, dstl, h)
    out = _tc_layer(p2, cnt, h, W2l, b2, W2r)
    return out[:N]
